# Initial kernel scaffold; baseline (speedup 1.0000x reference)
import jax
import jax.numpy as jnp
from jax.experimental import pallas as pl


def kernel(features, edge_index, edge_weight, weight):
    raise NotImplementedError("write your pallas kernel here")


# speedup vs baseline: 3.8243x; 3.8243x over previous
import functools

import jax
import jax.numpy as jnp
from jax import lax
from jax.experimental import pallas as pl
from jax.experimental.pallas import tpu as pltpu
from jax.experimental.pallas import tpu_sc as plsc

N_NODES = 10000
D_IN = 128
D_OUT = 128
N_EDGES = 320000

NC = 2
NS = 16
L = 16
NW = NC * NS

CH = 128
EPT = 10112
NCHUNK = EPT // CH
E_PAD = NW * EPT
ROWS_PER_TILE = N_NODES // NS


def _mm_body(f_ref, w_ref, o_ref):
    o_ref[...] = jnp.dot(f_ref[...], w_ref[...],
                         preferred_element_type=jnp.float32)


def _support_matmul(features, weight):
    blk = 1000
    grid = N_NODES // blk
    return pl.pallas_call(
        _mm_body,
        grid=(grid,),
        in_specs=[
            pl.BlockSpec((blk, D_IN), lambda i: (i, 0)),
            pl.BlockSpec((D_IN, D_OUT), lambda i: (0, 0)),
        ],
        out_specs=pl.BlockSpec((blk, D_OUT), lambda i: (i, 0)),
        out_shape=jax.ShapeDtypeStruct((N_NODES, D_OUT), jnp.float32),
    )(features, weight)


def _spmm_body(support_hbm, row_hbm, col_hbm, w_hbm, out_hbm,
               acc, col_v, row_v, w_v, gath_v, gsem):
    cid = lax.axis_index("c")
    sid = lax.axis_index("s")
    wid = sid * NC + cid

    @pl.loop(0, CH)
    def _zero(i):
        for j in range(D_OUT // L):
            gath_v[i, pl.ds(j * L, L)] = jnp.zeros((L,), jnp.float32)

    r0 = sid * ROWS_PER_TILE
    for k in range(5):
        pltpu.sync_copy(gath_v.at[pl.ds(0, 125)],
                        acc.at[pl.ds(r0 + k * 125, 125)])
    plsc.subcore_barrier()

    ebase = wid * EPT

    @pl.loop(0, NCHUNK)
    def _chunk(c):
        base = ebase + c * CH
        pltpu.sync_copy(col_hbm.at[pl.ds(base, CH)], col_v)
        pltpu.sync_copy(row_hbm.at[pl.ds(base, CH)], row_v)
        pltpu.sync_copy(w_hbm.at[pl.ds(base, CH)], w_v)
        pltpu.async_copy(support_hbm.at[col_v], gath_v, gsem).wait()

        @pl.loop(0, CH // L)
        def _scale(g):
            w16 = w_v[pl.ds(g * L, L)]
            for e in range(L):
                wb = jnp.full((L,), w16[e])
                for j in range(D_OUT // L):
                    sl = pl.ds(j * L, L)
                    gath_v[g * L + e, sl] = gath_v[g * L + e, sl] * wb

        pltpu.sync_copy(gath_v, acc.at[row_v], add=True)

    plsc.subcore_barrier()
    pltpu.sync_copy(acc.at[pl.ds(r0, ROWS_PER_TILE)], out_hbm.at[cid, sid])


def _sc_spmm(support, row, col, w):
    mesh = plsc.VectorSubcoreMesh(core_axis_name="c", subcore_axis_name="s")
    k = pl.kernel(
        _spmm_body,
        out_type=jax.ShapeDtypeStruct((NC, NS, ROWS_PER_TILE, D_OUT),
                                      jnp.float32),
        mesh=mesh,
        scratch_types=[
            pltpu.VMEM_SHARED((N_NODES, D_OUT), jnp.float32),
            pltpu.VMEM((CH,), jnp.int32),
            pltpu.VMEM((CH,), jnp.int32),
            pltpu.VMEM((CH,), jnp.float32),
            pltpu.VMEM((CH, D_OUT), jnp.float32),
            pltpu.SemaphoreType.DMA,
        ],
    )
    return k(support, row, col, w)


def _combine_body(p_ref, o_ref):
    o_ref[...] = jnp.maximum(p_ref[0] + p_ref[1], 0.0)


def _combine_relu(partials):
    blk = 1000
    grid = N_NODES // blk
    return pl.pallas_call(
        _combine_body,
        grid=(grid,),
        in_specs=[pl.BlockSpec((NC, blk, D_OUT), lambda i: (0, i, 0))],
        out_specs=pl.BlockSpec((blk, D_OUT), lambda i: (i, 0)),
        out_shape=jax.ShapeDtypeStruct((N_NODES, D_OUT), jnp.float32),
    )(partials)


def kernel(features, edge_index, edge_weight, weight):
    support = _support_matmul(features, weight)

    row = edge_index[0].astype(jnp.int32)
    col = edge_index[1].astype(jnp.int32)
    pad = E_PAD - N_EDGES
    row = jnp.pad(row, (0, pad))
    col = jnp.pad(col, (0, pad))
    w = jnp.pad(edge_weight, (0, pad))

    partials = _sc_spmm(support, row, col, w)
    partials = partials.reshape(NC, N_NODES, D_OUT)
    return _combine_relu(partials)
